# baseline (speedup 1.0000x reference)
import jax
import jax.numpy as jnp
from jax.experimental import pallas as pl
from jax.experimental.pallas import tpu as pltpu

_DIM = 128
_SIZE = 262144
_BLK = 8192


def _copy_body(src_ref, dst_ref):
    dst_ref[...] = src_ref[...]


def kernel(output, bank):
    bank_snapshot = pl.pallas_call(
        _copy_body,
        grid=(_SIZE // _BLK,),
        in_specs=[pl.BlockSpec((_DIM, _BLK), lambda i: (0, i))],
        out_specs=pl.BlockSpec((_DIM, _BLK), lambda i: (0, i)),
        out_shape=jax.ShapeDtypeStruct((_DIM, _SIZE), jnp.float32),
        compiler_params=pltpu.CompilerParams(
            dimension_semantics=("arbitrary",),
        ),
    )(bank)
    return (output, bank_snapshot)
